# Initial kernel scaffold; baseline (speedup 1.0000x reference)
import jax
import jax.numpy as jnp
from jax.experimental import pallas as pl


def kernel(responses, response_table, position_table):
    raise NotImplementedError("write your pallas kernel here")


# speedup vs baseline: 1.8449x; 1.8449x over previous
import functools

import jax
import jax.numpy as jnp
from jax import lax
from jax.experimental import pallas as pl
from jax.experimental.pallas import tpu as pltpu
from jax.experimental.pallas import tpu_sc as plsc

B = 4096
S = 200
D = 128
ROWS = B * S

NC = 2
NS = 16
NW = NC * NS
ROWS_W = ROWS // NW
CHUNK = 128
NCHUNK = ROWS_W // CHUNK


def _body(idx_hbm, tab_hbm, pos_hbm, out_hbm, idx_v, rows_v, pos2_v, sem):
    wid = lax.axis_index("s") * NC + lax.axis_index("c")
    wbase = wid * ROWS_W

    pltpu.sync_copy(pos_hbm, pos2_v.at[pl.ds(0, S)])
    pltpu.sync_copy(pos_hbm, pos2_v.at[pl.ds(S, S)])

    def chunk_step(c, _):
        g = wbase + c * CHUNK
        pltpu.sync_copy(idx_hbm.at[pl.ds(g, CHUNK)], idx_v)
        pltpu.async_copy(tab_hbm.at[idx_v], rows_v, sem).wait()
        phi = lax.rem(c * CHUNK, S)

        def row_step(r, _):
            for j in range(D // 16):
                sl = pl.ds(j * 16, 16)
                rows_v[r, sl] = rows_v[r, sl] + pos2_v[phi + r, sl]
            return 0

        lax.fori_loop(0, CHUNK, row_step, 0)
        pltpu.sync_copy(rows_v, out_hbm.at[pl.ds(g, CHUNK)])
        return 0

    lax.fori_loop(0, NCHUNK, chunk_step, 0)


@jax.jit
def _embed(idx_flat, response_table, position_table):
    mesh = plsc.VectorSubcoreMesh(core_axis_name="c", subcore_axis_name="s",
                                  num_cores=NC, num_subcores=NS)
    run = pl.kernel(
        _body,
        out_type=jax.ShapeDtypeStruct((ROWS, D), jnp.float32),
        mesh=mesh,
        scratch_types=[
            pltpu.VMEM((CHUNK,), jnp.int32),
            pltpu.VMEM((CHUNK, D), jnp.float32),
            pltpu.VMEM((2 * S, D), jnp.float32),
            pltpu.SemaphoreType.DMA,
        ],
    )
    return run(idx_flat, response_table, position_table)


def kernel(responses, response_table, position_table):
    idx_flat = responses.reshape(ROWS).astype(jnp.int32)
    out = _embed(idx_flat, response_table, position_table)
    return out.reshape(B, S, D)


# speedup vs baseline: 8.4773x; 4.5950x over previous
import jax
import jax.numpy as jnp
from jax import lax
from jax.experimental import pallas as pl
from jax.experimental.pallas import tpu as pltpu
from jax.experimental.pallas import tpu_sc as plsc

B = 4096
S = 200
D = 128
ROWS = B * S

NC = 2
NS = 16
NW = NC * NS
ROWS_W = ROWS // NW
CHUNK = 128
NCHUNK = ROWS_W // CHUNK
NSLOT = 3
POS_EXTRA = 120


def _body(idx_hbm, tab_hbm, pos_hbm, out_hbm, idx_v, rows_v, pos2_v, sem_g, sem_o):
    wid = lax.axis_index("s") * NC + lax.axis_index("c")
    wbase = wid * ROWS_W
    cbase = wid * NCHUNK

    pltpu.sync_copy(pos_hbm, pos2_v.at[pl.ds(0, S)])
    pltpu.sync_copy(pos_hbm.at[pl.ds(0, POS_EXTRA)], pos2_v.at[pl.ds(S, POS_EXTRA)])
    pltpu.sync_copy(idx_hbm.at[pl.ds(cbase, NCHUNK)], idx_v)

    def fire_gather(c):
        pltpu.async_copy(tab_hbm.at[idx_v.at[c]], rows_v.at[lax.rem(c, NSLOT)],
                         sem_g)

    fire_gather(0)
    fire_gather(1)

    def chunk_step(c, _):
        slot = lax.rem(c, NSLOT)
        g = wbase + c * CHUNK
        pltpu.make_async_copy(tab_hbm.at[pl.ds(0, CHUNK)], rows_v.at[slot],
                              sem_g).wait()
        phi = lax.rem(c * CHUNK, S)

        @plsc.parallel_loop(0, CHUNK, unroll=2)
        def row_step(r):
            for j in range(D // 16):
                sl = pl.ds(j * 16, 16)
                rows_v[slot, r, sl] = rows_v[slot, r, sl] + pos2_v[phi + r, sl]

        @pl.when(c >= 1)
        def _():
            pltpu.make_async_copy(
                rows_v.at[lax.rem(c - 1, NSLOT)],
                out_hbm.at[pl.ds(g - CHUNK, CHUNK)], sem_o).wait()

        pltpu.async_copy(rows_v.at[slot], out_hbm.at[pl.ds(g, CHUNK)], sem_o)

        @pl.when(c + 2 < NCHUNK)
        def _():
            fire_gather(c + 2)

        return 0

    lax.fori_loop(0, NCHUNK, chunk_step, 0)
    pltpu.make_async_copy(
        rows_v.at[lax.rem(NCHUNK - 1, NSLOT)],
        out_hbm.at[pl.ds(wbase + (NCHUNK - 1) * CHUNK, CHUNK)], sem_o).wait()


@jax.jit
def _embed(idx2d, response_table, position_table):
    mesh = plsc.VectorSubcoreMesh(core_axis_name="c", subcore_axis_name="s",
                                  num_cores=NC, num_subcores=NS)
    run = pl.kernel(
        _body,
        out_type=jax.ShapeDtypeStruct((ROWS, D), jnp.float32),
        mesh=mesh,
        scratch_types=[
            pltpu.VMEM((NCHUNK, CHUNK), jnp.int32),
            pltpu.VMEM((NSLOT, CHUNK, D), jnp.float32),
            pltpu.VMEM((S + POS_EXTRA, D), jnp.float32),
            pltpu.SemaphoreType.DMA,
            pltpu.SemaphoreType.DMA,
        ],
    )
    return run(idx2d, response_table, position_table)


def kernel(responses, response_table, position_table):
    idx2d = responses.reshape(ROWS // CHUNK, CHUNK).astype(jnp.int32)
    out = _embed(idx2d, response_table, position_table)
    return out.reshape(B, S, D)


# speedup vs baseline: 8.5140x; 1.0043x over previous
import jax
import jax.numpy as jnp
from jax import lax
from jax.experimental import pallas as pl
from jax.experimental.pallas import tpu as pltpu
from jax.experimental.pallas import tpu_sc as plsc

B = 4096
S = 200
D = 128
ROWS = B * S

NC = 2
NS = 16
NW = NC * NS
ROWS_W = ROWS // NW
CHUNK = 128
NCHUNK = ROWS_W // CHUNK
NSLOT = 3
POS_EXTRA = 120


def _body(idx_hbm, tab_hbm, pos_hbm, out_hbm, idx_v, rows_v, pos2_v, sem_g, sem_o):
    wid = lax.axis_index("s") * NC + lax.axis_index("c")
    wbase = wid * ROWS_W
    cbase = wid * NCHUNK

    pltpu.sync_copy(pos_hbm, pos2_v.at[pl.ds(0, S)])
    pltpu.sync_copy(pos_hbm.at[pl.ds(0, POS_EXTRA)], pos2_v.at[pl.ds(S, POS_EXTRA)])
    pltpu.sync_copy(idx_hbm.at[pl.ds(cbase, NCHUNK)], idx_v)

    def fire_gather(c):
        pltpu.async_copy(tab_hbm.at[idx_v.at[c]], rows_v.at[lax.rem(c, NSLOT)],
                         sem_g)

    fire_gather(0)
    fire_gather(1)

    def chunk_step(c, _):
        slot = lax.rem(c, NSLOT)
        g = wbase + c * CHUNK
        pltpu.make_async_copy(tab_hbm.at[pl.ds(0, CHUNK)], rows_v.at[slot],
                              sem_g).wait()
        phi = lax.rem(c * CHUNK, S)

        @plsc.parallel_loop(0, CHUNK, unroll=2)
        def row_step(r):
            for j in range(D // 16):
                sl = pl.ds(j * 16, 16)
                plsc.addupdate(rows_v.at[slot, r, sl], pos2_v[phi + r, sl])

        @pl.when(c >= 1)
        def _():
            pltpu.make_async_copy(
                rows_v.at[lax.rem(c - 1, NSLOT)],
                out_hbm.at[pl.ds(g - CHUNK, CHUNK)], sem_o).wait()

        pltpu.async_copy(rows_v.at[slot], out_hbm.at[pl.ds(g, CHUNK)], sem_o)

        @pl.when(c + 2 < NCHUNK)
        def _():
            fire_gather(c + 2)

        return 0

    lax.fori_loop(0, NCHUNK, chunk_step, 0)
    pltpu.make_async_copy(
        rows_v.at[lax.rem(NCHUNK - 1, NSLOT)],
        out_hbm.at[pl.ds(wbase + (NCHUNK - 1) * CHUNK, CHUNK)], sem_o).wait()


@jax.jit
def _embed(idx2d, response_table, position_table):
    mesh = plsc.VectorSubcoreMesh(core_axis_name="c", subcore_axis_name="s",
                                  num_cores=NC, num_subcores=NS)
    run = pl.kernel(
        _body,
        out_type=jax.ShapeDtypeStruct((ROWS, D), jnp.float32),
        mesh=mesh,
        scratch_types=[
            pltpu.VMEM((NCHUNK, CHUNK), jnp.int32),
            pltpu.VMEM((NSLOT, CHUNK, D), jnp.float32),
            pltpu.VMEM((S + POS_EXTRA, D), jnp.float32),
            pltpu.SemaphoreType.DMA,
            pltpu.SemaphoreType.DMA,
        ],
    )
    return run(idx2d, response_table, position_table)


def kernel(responses, response_table, position_table):
    idx2d = responses.reshape(ROWS // CHUNK, CHUNK).astype(jnp.int32)
    out = _embed(idx2d, response_table, position_table)
    return out.reshape(B, S, D)


# speedup vs baseline: 9.2024x; 1.0809x over previous
import jax
import jax.numpy as jnp
from jax import lax
from jax.experimental import pallas as pl
from jax.experimental.pallas import tpu as pltpu
from jax.experimental.pallas import tpu_sc as plsc

B = 4096
S = 200
D = 128
ROWS = B * S

NC = 2
NS = 16
NW = NC * NS
BAT_W = B // NW
CHUNK = S // 2
NHALF = 3
NIDX = 4


def _body(idx_hbm, tab_hbm, pos_hbm, out_hbm, idx_v, rows_v, pos_v,
          sem_i, sem_g, sem_o):
    wid = lax.axis_index("s") * NC + lax.axis_index("c")
    wbase = wid * BAT_W * S
    cbase = wid * 2 * BAT_W

    pltpu.sync_copy(pos_hbm, pos_v)

    def fire_idx(p):
        pltpu.async_copy(idx_hbm.at[pl.ds(cbase + 2 * p, 2)],
                         idx_v.at[lax.rem(p, NIDX)], sem_i)

    def wait_idx(p):
        pltpu.make_async_copy(idx_hbm.at[pl.ds(0, 2)],
                              idx_v.at[lax.rem(p, NIDX)], sem_i).wait()

    def fire_pair(p):
        half = lax.rem(p, NHALF)
        islot = lax.rem(p, NIDX)
        pltpu.async_copy(tab_hbm.at[idx_v.at[islot, 0]],
                         rows_v.at[pl.ds(half * S, CHUNK)], sem_g)
        pltpu.async_copy(tab_hbm.at[idx_v.at[islot, 1]],
                         rows_v.at[pl.ds(half * S + CHUNK, CHUNK)], sem_g)

    fire_idx(0)
    fire_idx(1)
    fire_idx(2)
    wait_idx(0)
    fire_pair(0)
    wait_idx(1)
    fire_pair(1)

    def batch_step(p, _):
        half = lax.rem(p, NHALF)
        base_v = half * S
        g = wbase + p * S

        @pl.when(p + 3 < BAT_W)
        def _():
            fire_idx(p + 3)

        pltpu.make_async_copy(tab_hbm.at[pl.ds(0, S)],
                              rows_v.at[pl.ds(base_v, S)], sem_g).wait()

        @plsc.parallel_loop(0, S, unroll=2)
        def row_step(r):
            for j in range(D // 16):
                sl = pl.ds(j * 16, 16)
                plsc.addupdate(rows_v.at[base_v + r, sl], pos_v[r, sl])

        @pl.when(p >= 1)
        def _():
            pltpu.make_async_copy(
                rows_v.at[pl.ds(lax.rem(p - 1, NHALF) * S, S)],
                out_hbm.at[pl.ds(g - S, S)], sem_o).wait()

        pltpu.async_copy(rows_v.at[pl.ds(base_v, S)],
                         out_hbm.at[pl.ds(g, S)], sem_o)

        @pl.when(p + 2 < BAT_W)
        def _():
            wait_idx(p + 2)
            fire_pair(p + 2)

        return 0

    lax.fori_loop(0, BAT_W, batch_step, 0)
    pltpu.make_async_copy(
        rows_v.at[pl.ds(lax.rem(BAT_W - 1, NHALF) * S, S)],
        out_hbm.at[pl.ds(wbase + (BAT_W - 1) * S, S)], sem_o).wait()


@jax.jit
def _embed(idx2d, response_table, position_table):
    mesh = plsc.VectorSubcoreMesh(core_axis_name="c", subcore_axis_name="s",
                                  num_cores=NC, num_subcores=NS)
    run = pl.kernel(
        _body,
        out_type=jax.ShapeDtypeStruct((ROWS, D), jnp.float32),
        mesh=mesh,
        scratch_types=[
            pltpu.VMEM((NIDX, 2, CHUNK), jnp.int32),
            pltpu.VMEM((NHALF * S, D), jnp.float32),
            pltpu.VMEM((S, D), jnp.float32),
            pltpu.SemaphoreType.DMA,
            pltpu.SemaphoreType.DMA,
            pltpu.SemaphoreType.DMA,
        ],
    )
    return run(idx2d, response_table, position_table)


def kernel(responses, response_table, position_table):
    idx2d = responses.reshape(ROWS // CHUNK, CHUNK).astype(jnp.int32)
    out = _embed(idx2d, response_table, position_table)
    return out.reshape(B, S, D)
